# Initial kernel scaffold; baseline (speedup 1.0000x reference)
import jax
import jax.numpy as jnp
from jax.experimental import pallas as pl


def kernel(x_hero, x_enemy, x_bullet, x_door, x_wall, ei_enemy, ei_bullet, ei_door, ei_wall, batch, Wl_enemy, bl_enemy, Wr_enemy, Wl_bullet, bl_bullet, Wr_bullet, Wl_door, bl_door, Wr_door, Wl_wall, bl_wall, Wr_wall, W_fc, b_fc):
    raise NotImplementedError("write your pallas kernel here")


# speedup vs baseline: 23.8977x; 23.8977x over previous
import functools

import jax
import jax.numpy as jnp
from jax import lax
from jax.experimental import pallas as pl
from jax.experimental.pallas import tpu as pltpu
from jax.experimental.pallas import tpu_sc as plsc

_N = 50000
_E = 800000
_NG = 1024
_HID = 512

_NC = 2
_NS = 16
_G = 256
_GROUPS = 3200
_EPAD = _GROUPS * _G
_GPT = _GROUPS // _NS
_NACC = 51200
_RPT = _NACC // _NS
_TPT = _N // _NS
_BN = 2048
_NB = _NACC // _BN
_CH = 20
_NBUF = 5
_DIMS = (4, 3, 4, 4)


def _sc_body(xe, xb, xd, xw, se, de, sb, db, sd, dd, sw, dw, zz,
             out, tab0, tab1, acc0, acc1, sidx, didx, rows, sem):
    c = lax.axis_index("c")
    s = lax.axis_index("s")
    t0 = s * _RPT
    u0 = s * _TPT

    pltpu.sync_copy(zz, acc0.at[pl.ds(t0, _RPT)])
    pltpu.sync_copy(zz, acc1.at[pl.ds(t0, _RPT)])

    @pl.when(c == 0)
    def _():
        pltpu.sync_copy(xe.at[pl.ds(u0, _TPT)], tab0.at[pl.ds(u0, _TPT)])
        pltpu.sync_copy(xb.at[pl.ds(u0, _TPT)], tab1.at[pl.ds(u0, _TPT)])

    @pl.when(c == 1)
    def _():
        pltpu.sync_copy(xd.at[pl.ds(u0, _TPT)], tab0.at[pl.ds(u0, _TPT)])
        pltpu.sync_copy(xw.at[pl.ds(u0, _TPT)], tab1.at[pl.ds(u0, _TPT)])

    plsc.subcore_barrier()

    g0 = s * _GPT

    def process(src_h, dst_h, tab, acc):
        for ch in range(_GPT // _CH):
            pltpu.sync_copy(src_h.at[pl.ds(g0 + ch * _CH, _CH)], sidx)
            pltpu.sync_copy(dst_h.at[pl.ds(g0 + ch * _CH, _CH)], didx)
            for b in range(_NBUF):
                pltpu.async_copy(tab.at[sidx.at[b]], rows.at[b], sem.at[b])

            def body(p, _):
                for b in range(_NBUF):
                    j = p * _NBUF + b
                    pltpu.make_async_copy(
                        tab.at[sidx.at[j]], rows.at[b], sem.at[b]).wait()
                    pltpu.sync_copy(rows.at[b], acc.at[didx.at[j]], add=True)

                    @pl.when(j + _NBUF < _CH)
                    def _(b=b, j=j):
                        pltpu.async_copy(
                            tab.at[sidx.at[j + _NBUF]], rows.at[b], sem.at[b])
                return 0

            lax.fori_loop(0, _CH // _NBUF, body, 0)

    @pl.when(c == 0)
    def _():
        process(se, de, tab0, acc0)
        process(sb, db, tab1, acc1)

    @pl.when(c == 1)
    def _():
        process(sd, dd, tab0, acc0)
        process(sw, dw, tab1, acc1)

    plsc.subcore_barrier()
    pltpu.sync_copy(acc0.at[pl.ds(t0, _RPT)],
                    out.at[2 * c, pl.ds(t0, _RPT)])
    pltpu.sync_copy(acc1.at[pl.ds(t0, _RPT)],
                    out.at[2 * c + 1, pl.ds(t0, _RPT)])


@functools.cache
def _get_sc_edge_agg():
    mesh = plsc.VectorSubcoreMesh(core_axis_name="c", subcore_axis_name="s",
                                  num_cores=_NC, num_subcores=_NS)
    return pl.kernel(
        _sc_body,
        out_type=jax.ShapeDtypeStruct((4, _NACC, 8), jnp.float32),
        mesh=mesh,
        scratch_types=[
            pltpu.VMEM_SHARED((_N, 8), jnp.float32),
            pltpu.VMEM_SHARED((_N, 8), jnp.float32),
            pltpu.VMEM_SHARED((_NACC, 8), jnp.float32),
            pltpu.VMEM_SHARED((_NACC, 8), jnp.float32),
            pltpu.VMEM((_CH, _G), jnp.int32),
            pltpu.VMEM((_CH, _G), jnp.int32),
            pltpu.VMEM((_NBUF, _G, 8), jnp.float32),
            pltpu.SemaphoreType.DMA((_NBUF,)),
        ],
        compiler_params=pltpu.CompilerParams(use_tc_tiling_on_sc=False),
    )


def _tc_body(parts, xh, b3, wle, wlb, wld, wlw, wrs, bls, wfc, bfc,
             out, acc):
    i = pl.program_id(0)
    zs = []
    for r in range(4):
        a = parts[r]
        d = _DIMS[r]
        cnt = a[:, d:d + 1]
        zs.append(a[:, :d] / jnp.maximum(cnt, 1.0))
    zs.append(xh[:, :6])
    zs.append(jnp.ones((_BN, 1), jnp.float32))
    zs.append(jnp.zeros((_BN, 2), jnp.float32))
    z = jnp.concatenate(zs, axis=1)

    b = b3[0]
    ohT = (lax.broadcasted_iota(jnp.int32, (_NG, _BN), 0) == b)
    ohT = ohT.astype(jnp.float32)
    partial = lax.dot_general(ohT, z, (((1,), (0,)), ((), ())),
                              precision=lax.Precision.HIGHEST,
                              preferred_element_type=jnp.float32)

    @pl.when(i == 0)
    def _():
        acc[...] = jnp.zeros_like(acc)

    acc[...] += partial

    @pl.when(i == _NB - 1)
    def _():
        s = acc[...]
        gcnt = s[:, 21:22]
        m = s / jnp.maximum(gcnt, 1.0)

        def dot(a, b):
            return lax.dot_general(a, b, (((1,), (0,)), ((), ())),
                                   precision=lax.Precision.HIGHEST,
                                   preferred_element_type=jnp.float32)

        h = (dot(m[:, 0:4], wle[...]) + dot(m[:, 4:7], wlb[...])
             + dot(m[:, 7:11], wld[...]) + dot(m[:, 11:15], wlw[...])
             + dot(m[:, 15:21], wrs[...]) + bls[...])
        o = dot(h, wfc[...]) + bfc[...]
        out[...] = jnp.where(gcnt > 0.0, o, bfc[...])


_tc_pool = pl.pallas_call(
    _tc_body,
    grid=(_NB,),
    in_specs=[
        pl.BlockSpec((4, _BN, 8), lambda i: (0, i, 0)),
        pl.BlockSpec((_BN, 8), lambda i: (i, 0)),
        pl.BlockSpec((1, 1, _BN), lambda i: (i, 0, 0)),
        pl.BlockSpec((4, _HID), lambda i: (0, 0)),
        pl.BlockSpec((3, _HID), lambda i: (0, 0)),
        pl.BlockSpec((4, _HID), lambda i: (0, 0)),
        pl.BlockSpec((4, _HID), lambda i: (0, 0)),
        pl.BlockSpec((6, _HID), lambda i: (0, 0)),
        pl.BlockSpec((1, _HID), lambda i: (0, 0)),
        pl.BlockSpec((_HID, 9), lambda i: (0, 0)),
        pl.BlockSpec((1, 9), lambda i: (0, 0)),
    ],
    out_specs=pl.BlockSpec((_NG, 9), lambda i: (0, 0)),
    out_shape=jax.ShapeDtypeStruct((_NG, 9), jnp.float32),
    scratch_shapes=[pltpu.VMEM((_NG, 24), jnp.float32)],
)


def _prep_ei(ei):
    pad = _EPAD - _E
    src = jnp.concatenate([ei[0], jnp.zeros((pad,), jnp.int32)])
    dst = jnp.concatenate([ei[1], jnp.full((pad,), _N, jnp.int32)])
    return src.reshape(_GROUPS, _G), dst.reshape(_GROUPS, _G)


def kernel(x_hero, x_enemy, x_bullet, x_door, x_wall,
           ei_enemy, ei_bullet, ei_door, ei_wall, batch,
           Wl_enemy, bl_enemy, Wr_enemy,
           Wl_bullet, bl_bullet, Wr_bullet,
           Wl_door, bl_door, Wr_door,
           Wl_wall, bl_wall, Wr_wall,
           W_fc, b_fc):
    f32 = jnp.float32
    xps = []
    for x, d in ((x_enemy, 4), (x_bullet, 3), (x_door, 4), (x_wall, 4)):
        xps.append(jnp.concatenate(
            [x, jnp.ones((_N, 1), f32), jnp.zeros((_N, 7 - d), f32)], axis=1))
    se, de = _prep_ei(ei_enemy)
    sb, db = _prep_ei(ei_bullet)
    sd, dd = _prep_ei(ei_door)
    sw, dw = _prep_ei(ei_wall)
    zz = jnp.zeros((_RPT, 8), f32)

    parts = _get_sc_edge_agg()(xps[0], xps[1], xps[2], xps[3],
                               se, de, sb, db, sd, dd, sw, dw, zz)

    xh = jnp.zeros((_NACC, 8), f32).at[:_N, :6].set(x_hero)
    b3 = jnp.concatenate(
        [batch, jnp.full((_NACC - _N,), _NG, jnp.int32)]).reshape(_NB, 1, _BN)
    wrs = Wr_enemy + Wr_bullet + Wr_door + Wr_wall
    bls = (bl_enemy + bl_bullet + bl_door + bl_wall).reshape(1, _HID)

    return _tc_pool(parts, xh, b3, Wl_enemy, Wl_bullet, Wl_door, Wl_wall,
                    wrs, bls, W_fc, b_fc.reshape(1, 9))


# speedup vs baseline: 27.9226x; 1.1684x over previous
import functools

import jax
import jax.numpy as jnp
from jax import lax
from jax.experimental import pallas as pl
from jax.experimental.pallas import tpu as pltpu
from jax.experimental.pallas import tpu_sc as plsc

_N = 50000
_E = 800000
_NG = 1024
_HID = 512

_NC = 2
_NS = 16
_G = 256
_GROUPS = 3200
_EPAD = _GROUPS * _G
_GPT = _GROUPS // _NS
_NACC = 51200
_RPT = _NACC // _NS
_TPT = _N // _NS
_NTAB = 51200
_ZPT = (_NTAB - _N) // _NS
_BN = 2048
_NB = _NACC // _BN
_CH = 20
_NBUF = 5
_DIMS = (4, 3, 4, 4)


def _sc_body(xe, xb, xd, xw, pe, pb, pd, pw, zz,
             out, tab0, tab1, acc0, acc1, sidx, didx, rows, sem):
    c = lax.axis_index("c")
    s = lax.axis_index("s")
    t0 = s * _RPT
    u0 = s * _TPT

    pltpu.sync_copy(zz, acc0.at[pl.ds(t0, _RPT)])
    pltpu.sync_copy(zz, acc1.at[pl.ds(t0, _RPT)])
    z0 = _N + s * _ZPT
    pltpu.sync_copy(zz.at[pl.ds(0, _ZPT)], tab0.at[pl.ds(z0, _ZPT)])
    pltpu.sync_copy(zz.at[pl.ds(0, _ZPT)], tab1.at[pl.ds(z0, _ZPT)])

    @pl.when(c == 0)
    def _():
        pltpu.sync_copy(xe.at[pl.ds(u0, _TPT)], tab0.at[pl.ds(u0, _TPT)])
        pltpu.sync_copy(xb.at[pl.ds(u0, _TPT)], tab1.at[pl.ds(u0, _TPT)])

    @pl.when(c == 1)
    def _():
        pltpu.sync_copy(xd.at[pl.ds(u0, _TPT)], tab0.at[pl.ds(u0, _TPT)])
        pltpu.sync_copy(xw.at[pl.ds(u0, _TPT)], tab1.at[pl.ds(u0, _TPT)])

    plsc.subcore_barrier()

    g0 = s * _GPT

    def process(ei_h, tab, acc):
        for ch in range(_GPT // _CH):
            pltpu.sync_copy(ei_h.at[0, pl.ds(g0 + ch * _CH, _CH)], sidx)
            pltpu.sync_copy(ei_h.at[1, pl.ds(g0 + ch * _CH, _CH)], didx)
            for b in range(_NBUF):
                pltpu.async_copy(tab.at[sidx.at[b]], rows.at[b], sem.at[b])

            def body(p, _):
                for b in range(_NBUF):
                    j = p * _NBUF + b
                    pltpu.make_async_copy(
                        tab.at[sidx.at[j]], rows.at[b], sem.at[b]).wait()
                    pltpu.sync_copy(rows.at[b], acc.at[didx.at[j]], add=True)

                    @pl.when(j + _NBUF < _CH)
                    def _(b=b, j=j):
                        pltpu.async_copy(
                            tab.at[sidx.at[j + _NBUF]], rows.at[b], sem.at[b])
                return 0

            lax.fori_loop(0, _CH // _NBUF, body, 0)

    @pl.when(c == 0)
    def _():
        process(pe, tab0, acc0)
        process(pb, tab1, acc1)

    @pl.when(c == 1)
    def _():
        process(pd, tab0, acc0)
        process(pw, tab1, acc1)

    plsc.subcore_barrier()
    pltpu.sync_copy(acc0.at[pl.ds(t0, _RPT)],
                    out.at[2 * c, pl.ds(t0, _RPT)])
    pltpu.sync_copy(acc1.at[pl.ds(t0, _RPT)],
                    out.at[2 * c + 1, pl.ds(t0, _RPT)])


@functools.cache
def _get_sc_edge_agg():
    mesh = plsc.VectorSubcoreMesh(core_axis_name="c", subcore_axis_name="s",
                                  num_cores=_NC, num_subcores=_NS)
    return pl.kernel(
        _sc_body,
        out_type=jax.ShapeDtypeStruct((4, _NACC, 8), jnp.float32),
        mesh=mesh,
        scratch_types=[
            pltpu.VMEM_SHARED((_NTAB, 8), jnp.float32),
            pltpu.VMEM_SHARED((_NTAB, 8), jnp.float32),
            pltpu.VMEM_SHARED((_NACC, 8), jnp.float32),
            pltpu.VMEM_SHARED((_NACC, 8), jnp.float32),
            pltpu.VMEM((_CH, _G), jnp.int32),
            pltpu.VMEM((_CH, _G), jnp.int32),
            pltpu.VMEM((_NBUF, _G, 8), jnp.float32),
            pltpu.SemaphoreType.DMA((_NBUF,)),
        ],
        compiler_params=pltpu.CompilerParams(use_tc_tiling_on_sc=False),
    )


def _tc_body(parts, xh, b3, wle, wlb, wld, wlw, wrs, bls, wfc, bfc,
             out, acc):
    i = pl.program_id(0)
    zs = []
    for r in range(4):
        a = parts[r]
        d = _DIMS[r]
        cnt = a[:, d:d + 1]
        zs.append(a[:, :d] / jnp.maximum(cnt, 1.0))
    zs.append(xh[:, :6])
    zs.append(jnp.ones((_BN, 1), jnp.float32))
    zs.append(jnp.zeros((_BN, 2), jnp.float32))
    z = jnp.concatenate(zs, axis=1)

    b = b3[0]
    ohT = (lax.broadcasted_iota(jnp.int32, (_NG, _BN), 0) == b)
    ohT = ohT.astype(jnp.float32)
    partial = lax.dot_general(ohT, z, (((1,), (0,)), ((), ())),
                              precision=lax.Precision.HIGHEST,
                              preferred_element_type=jnp.float32)

    @pl.when(i == 0)
    def _():
        acc[...] = jnp.zeros_like(acc)

    acc[...] += partial

    @pl.when(i == _NB - 1)
    def _():
        s = acc[...]
        gcnt = s[:, 21:22]
        m = s / jnp.maximum(gcnt, 1.0)

        def dot(a, b):
            return lax.dot_general(a, b, (((1,), (0,)), ((), ())),
                                   precision=lax.Precision.HIGHEST,
                                   preferred_element_type=jnp.float32)

        h = (dot(m[:, 0:4], wle[...]) + dot(m[:, 4:7], wlb[...])
             + dot(m[:, 7:11], wld[...]) + dot(m[:, 11:15], wlw[...])
             + dot(m[:, 15:21], wrs[...]) + bls[...])
        o = dot(h, wfc[...]) + bfc[...]
        out[...] = jnp.where(gcnt > 0.0, o, bfc[...])


_tc_pool = pl.pallas_call(
    _tc_body,
    grid=(_NB,),
    in_specs=[
        pl.BlockSpec((4, _BN, 8), lambda i: (0, i, 0)),
        pl.BlockSpec((_BN, 8), lambda i: (i, 0)),
        pl.BlockSpec((1, 1, _BN), lambda i: (i, 0, 0)),
        pl.BlockSpec((4, _HID), lambda i: (0, 0)),
        pl.BlockSpec((3, _HID), lambda i: (0, 0)),
        pl.BlockSpec((4, _HID), lambda i: (0, 0)),
        pl.BlockSpec((4, _HID), lambda i: (0, 0)),
        pl.BlockSpec((6, _HID), lambda i: (0, 0)),
        pl.BlockSpec((1, _HID), lambda i: (0, 0)),
        pl.BlockSpec((_HID, 9), lambda i: (0, 0)),
        pl.BlockSpec((1, 9), lambda i: (0, 0)),
    ],
    out_specs=pl.BlockSpec((_NG, 9), lambda i: (0, 0)),
    out_shape=jax.ShapeDtypeStruct((_NG, 9), jnp.float32),
    scratch_shapes=[pltpu.VMEM((_NG, 24), jnp.float32)],
)


def _prep_ei(ei):
    return jnp.pad(ei, ((0, 0), (0, _EPAD - _E)),
                   constant_values=_N).reshape(2, _GROUPS, _G)


def kernel(x_hero, x_enemy, x_bullet, x_door, x_wall,
           ei_enemy, ei_bullet, ei_door, ei_wall, batch,
           Wl_enemy, bl_enemy, Wr_enemy,
           Wl_bullet, bl_bullet, Wr_bullet,
           Wl_door, bl_door, Wr_door,
           Wl_wall, bl_wall, Wr_wall,
           W_fc, b_fc):
    f32 = jnp.float32
    xps = []
    for x, d in ((x_enemy, 4), (x_bullet, 3), (x_door, 4), (x_wall, 4)):
        xps.append(jnp.concatenate(
            [x, jnp.ones((_N, 1), f32), jnp.zeros((_N, 7 - d), f32)], axis=1))
    pe = _prep_ei(ei_enemy)
    pb = _prep_ei(ei_bullet)
    pd = _prep_ei(ei_door)
    pw = _prep_ei(ei_wall)
    zz = jnp.zeros((_RPT, 8), f32)

    parts = _get_sc_edge_agg()(xps[0], xps[1], xps[2], xps[3],
                               pe, pb, pd, pw, zz)

    xh = jnp.zeros((_NACC, 8), f32).at[:_N, :6].set(x_hero)
    b3 = jnp.concatenate(
        [batch, jnp.full((_NACC - _N,), _NG, jnp.int32)]).reshape(_NB, 1, _BN)
    wrs = Wr_enemy + Wr_bullet + Wr_door + Wr_wall
    bls = (bl_enemy + bl_bullet + bl_door + bl_wall).reshape(1, _HID)

    return _tc_pool(parts, xh, b3, Wl_enemy, Wl_bullet, Wl_door, Wl_wall,
                    wrs, bls, W_fc, b_fc.reshape(1, 9))
